# Initial kernel scaffold; baseline (speedup 1.0000x reference)
import jax
import jax.numpy as jnp
from jax.experimental import pallas as pl


def kernel(predict_latent, k, weight, embeds):
    raise NotImplementedError("write your pallas kernel here")


# speedup vs baseline: 15.5621x; 15.5621x over previous
import jax
import jax.numpy as jnp
from jax.experimental import pallas as pl
from jax.experimental.pallas import tpu as pltpu

_B = 4096
_D = 128
_C = 512


def _dist_kernel(x_ref, k_ref, w_ref, e_ref, out_ref):
    w = w_ref[0, :]
    m = jnp.mean(w)
    var = jnp.sum((w - m) ** 2) * (1.0 / (_D - 1))
    wn = (w - m) / jnp.sqrt(var + 1e-5)
    ww = 1.0 / (1.0 + jnp.exp(-7.0 * wn))

    xw = x_ref[:, :] * ww[None, :]
    ew = e_ref[:, :] * ww[None, :]
    xx = jnp.sum(xw * xw, axis=1, keepdims=True)
    ones = jnp.ones((1, _D), dtype=jnp.float32)
    ee = jax.lax.dot_general(ones, ew * ew,
                             (((1,), (1,)), ((), ())),
                             preferred_element_type=jnp.float32)
    cross = jax.lax.dot_general(xw, ew,
                                (((1,), (1,)), ((), ())),
                                preferred_element_type=jnp.float32)
    all_dist = (xx + ee - 2.0 * cross) * (1.0 / _D)

    cols = jax.lax.broadcasted_iota(jnp.int32, (x_ref.shape[0], _C), 1)
    onehot = cols == k_ref[:, :]
    pd = jnp.sum(jnp.where(onehot, all_dist, 0.0), axis=1, keepdims=True)

    diff = pd - all_dist
    mask = diff >= 0.0
    out_ref[0] = jnp.sum(pd)
    out_ref[1] = jnp.sum(jnp.where(mask, diff, 0.0))
    out_ref[2] = jnp.sum(mask.astype(jnp.float32))


def kernel(predict_latent, k, weight, embeds):
    k2 = k.astype(jnp.int32).reshape(_B, 1)
    w2 = weight.reshape(1, _D)
    partials = pl.pallas_call(
        _dist_kernel,
        out_shape=jax.ShapeDtypeStruct((3,), jnp.float32),
        out_specs=pl.BlockSpec(memory_space=pltpu.SMEM),
    )(predict_latent, k2, w2, embeds)
    mean_pdist = partials[0] / _B
    diffargmin = partials[1] / partials[2]
    return (mean_pdist, diffargmin)


# speedup vs baseline: 20.5435x; 1.3201x over previous
import jax
import jax.numpy as jnp
from jax.experimental import pallas as pl
from jax.experimental.pallas import tpu as pltpu

_B = 4096
_D = 128
_C = 512
_NB = 8
_BM = _B // _NB


def _dist_kernel(x_ref, k_ref, w_ref, e_ref, out_ref,
                 ww_ref, ew2_ref, ees_ref, acc_ref):
    i = pl.program_id(0)
    s = 1.0 / _D

    @pl.when(i == 0)
    def _prep():
        w = w_ref[0, :]
        m = jnp.mean(w)
        var = jnp.sum((w - m) ** 2) * (1.0 / (_D - 1))
        wn = (w - m) * jax.lax.rsqrt(var + 1e-5)
        ww = 1.0 / (1.0 + jnp.exp(-7.0 * wn))
        ww_ref[0, :] = ww
        ew = e_ref[:, :] * ww[None, :]
        ew2_ref[:, :] = ew * (2.0 * s)
        ones = jnp.ones((1, _D), dtype=jnp.float32)
        ees_ref[0, :] = jax.lax.dot_general(
            ones, ew * ew, (((1,), (1,)), ((), ())),
            preferred_element_type=jnp.float32)[0, :] * s
        acc_ref[0] = 0.0
        acc_ref[1] = 0.0
        acc_ref[2] = 0.0

    ww = ww_ref[0, :]
    xw = x_ref[:, :] * ww[None, :]
    xxs = jnp.sum(xw * xw, axis=1, keepdims=True) * s
    cross2 = jax.lax.dot_general(
        xw, ew2_ref[:, :], (((1,), (1,)), ((), ())),
        preferred_element_type=jnp.float32)
    all_dist = (xxs + ees_ref[0, :][None, :]) - cross2

    cols = jax.lax.broadcasted_iota(jnp.int32, (_BM, _C), 1)
    pd = jnp.sum(jnp.where(cols == k_ref[:, :], all_dist, 0.0),
                 axis=1, keepdims=True)
    r = pd - all_dist
    acc_ref[0] = acc_ref[0] + jnp.sum(pd)
    acc_ref[1] = acc_ref[1] + jnp.sum(jnp.maximum(r, 0.0))
    acc_ref[2] = acc_ref[2] + jnp.sum((r >= 0.0).astype(jnp.float32))

    @pl.when(i == _NB - 1)
    def _fin():
        out_ref[0] = acc_ref[0] * (1.0 / _B)
        out_ref[1] = acc_ref[1] / acc_ref[2]


def kernel(predict_latent, k, weight, embeds):
    k2 = k.astype(jnp.int32).reshape(_B, 1)
    w2 = weight.reshape(1, _D)
    out = pl.pallas_call(
        _dist_kernel,
        grid=(_NB,),
        in_specs=[
            pl.BlockSpec((_BM, _D), lambda i: (i, 0)),
            pl.BlockSpec((_BM, 1), lambda i: (i, 0)),
            pl.BlockSpec((1, _D), lambda i: (0, 0)),
            pl.BlockSpec((_C, _D), lambda i: (0, 0)),
        ],
        out_specs=pl.BlockSpec(memory_space=pltpu.SMEM),
        out_shape=jax.ShapeDtypeStruct((2,), jnp.float32),
        scratch_shapes=[
            pltpu.VMEM((1, _D), jnp.float32),
            pltpu.VMEM((_C, _D), jnp.float32),
            pltpu.VMEM((1, _C), jnp.float32),
            pltpu.SMEM((3,), jnp.float32),
        ],
    )(predict_latent, k2, w2, embeds)
    return (out[0], out[1])
